# baseline (speedup 1.0000x reference)
import functools

import jax
import jax.numpy as jnp
from jax import lax
from jax.experimental import pallas as pl
from jax.experimental.pallas import tpu as pltpu
from jax.experimental.pallas import tpu_sc as plsc

H, W, NPF = 32, 32, 384
LANES = 16
NC, NS = 2, 16
SUBROWS = 2 * W


@functools.partial(
    pl.kernel,
    out_type=(
        jax.ShapeDtypeStruct((H * W * 2, NPF), jnp.float32),
        jax.ShapeDtypeStruct((NC * (H + W), NPF), jnp.float32),
    ),
    mesh=plsc.VectorSubcoreMesh(core_axis_name="c", subcore_axis_name="s"),
    scratch_types=[
        pltpu.VMEM((SUBROWS, NPF), jnp.float32),
        pltpu.VMEM((SUBROWS,), jnp.int32),
        pltpu.SemaphoreType.DMA,
    ],
)
def _pos_embed_sc(row_hbm, col_hbm, out_hbm, tbl_hbm, buf, idx_v, sem):
    c = lax.axis_index("c")
    s = lax.axis_index("s")
    h = s * NC + c

    @pl.when(s == 0)
    def _():
        pltpu.sync_copy(col_hbm, tbl_hbm.at[pl.ds(c * (H + W), W), :])

    @pl.when(s == 1)
    def _():
        pltpu.sync_copy(row_hbm, tbl_hbm.at[pl.ds(c * (H + W) + W, H), :])

    base = c * (H + W)
    for v in range(SUBROWS // LANES):
        j = lax.iota(jnp.int32, LANES) + (v * LANES)
        idx = jnp.where((j & 1) == 1, W + h, j >> 1) + base
        idx_v[pl.ds(v * LANES, LANES)] = idx

    plsc.subcore_barrier()

    pltpu.async_copy(tbl_hbm.at[idx_v], buf, sem).wait()
    pltpu.sync_copy(buf, out_hbm.at[pl.ds(h * SUBROWS, SUBROWS), :])


def kernel(tensor, row_embed, col_embed):
    del tensor
    pos, _ = _pos_embed_sc(row_embed, col_embed)
    return pos.reshape(H * W, 2 * NPF)


# speedup vs baseline: 1.4000x; 1.4000x over previous
import functools

import jax
import jax.numpy as jnp
from jax import lax
from jax.experimental import pallas as pl
from jax.experimental.pallas import tpu as pltpu
from jax.experimental.pallas import tpu_sc as plsc

H, W, NPF = 32, 32, 384
LANES = 16
NREG = NPF // LANES
NC, NS = 2, 16
NCHUNK = 4
RPC = W // NCHUNK


@functools.partial(
    pl.kernel,
    out_type=jax.ShapeDtypeStruct((H * W, 2 * NPF), jnp.float32),
    mesh=plsc.VectorSubcoreMesh(core_axis_name="c", subcore_axis_name="s"),
    scratch_types=[
        pltpu.VMEM((W, 2 * NPF), jnp.float32),
        pltpu.VMEM((NPF,), jnp.float32),
        pltpu.SemaphoreType.DMA,
        [pltpu.SemaphoreType.DMA] * NCHUNK,
        [pltpu.SemaphoreType.DMA] * NCHUNK,
    ],
)
def _pos_embed_sc(row_hbm, col_hbm, out_hbm, buf, row_v, sem_row, sems_col,
                  sems_out):
    h = lax.axis_index("s") * NC + lax.axis_index("c")

    cps_col = []
    for c in range(NCHUNK):
        cp = pltpu.make_async_copy(
            col_hbm.at[pl.ds(c * RPC, RPC), :],
            buf.at[pl.ds(c * RPC, RPC), pl.ds(0, NPF)],
            sems_col[c])
        cp.start()
        cps_col.append(cp)

    cp_row = pltpu.make_async_copy(row_hbm.at[h], row_v, sem_row)
    cp_row.start()
    cp_row.wait()
    regs = [row_v[pl.ds(LANES * i, LANES)] for i in range(NREG)]

    def fill_row(r, carry):
        for i in range(NREG):
            buf[r, pl.ds(NPF + LANES * i, LANES)] = regs[i]
        return carry

    cps_out = []
    for c in range(NCHUNK):
        lax.fori_loop(c * RPC, (c + 1) * RPC, fill_row, 0, unroll=4)
        cps_col[c].wait()
        cp = pltpu.make_async_copy(
            buf.at[pl.ds(c * RPC, RPC), :],
            out_hbm.at[pl.ds(h * W + c * RPC, RPC), :],
            sems_out[c])
        cp.start()
        cps_out.append(cp)
    for cp in cps_out:
        cp.wait()


def kernel(tensor, row_embed, col_embed):
    del tensor
    return _pos_embed_sc(row_embed, col_embed)


# speedup vs baseline: 13.8222x; 9.8731x over previous
import jax
import jax.numpy as jnp
from jax.experimental import pallas as pl
from jax.experimental.pallas import tpu as pltpu

H, W, NPF = 32, 32, 384


def _body(row_ref, col_ref, out_ref):
    colb = jnp.broadcast_to(col_ref[...][None, :, :], (H, W, NPF))
    rowb = jnp.broadcast_to(row_ref[...][:, None, :], (H, W, NPF))
    out_ref[:, : NPF] = colb.reshape(H * W, NPF)
    out_ref[:, NPF:] = rowb.reshape(H * W, NPF)


def kernel(tensor, row_embed, col_embed):
    del tensor
    return pl.pallas_call(
        _body,
        out_shape=jax.ShapeDtypeStruct((H * W, 2 * NPF), jnp.float32),
    )(row_embed, col_embed)
